# baseline (speedup 1.0000x reference)
import functools

import jax
import jax.numpy as jnp
from jax import lax
from jax.experimental import pallas as pl
from jax.experimental.pallas import tpu as pltpu, tpu_sc as plsc


def _make_gather(V, D, B):
  info = plsc.get_sparse_core_info()
  NC, NS = info.num_cores, info.num_subcores
  NW = NC * NS
  assert B % NW == 0
  b_per_w = B // NW
  C = 200
  NBUF = 4
  K = 3
  assert b_per_w % C == 0
  n_chunks = b_per_w // C
  n_tail = n_chunks % NBUF
  n_rounds = n_chunks // NBUF

  mesh = plsc.VectorSubcoreMesh(core_axis_name="c", subcore_axis_name="s")

  scratch = ([pltpu.VMEM((b_per_w,), jnp.int32)]
             + [pltpu.VMEM((C, D), jnp.float32)] * NBUF
             + [pltpu.SemaphoreType.DMA] * (2 * NBUF + 1))

  @functools.partial(
      pl.kernel,
      mesh=mesh,
      out_type=jax.ShapeDtypeStruct((B, D), jnp.float32),
      scratch_types=scratch,
  )
  def k(table_hbm, idx_hbm, out_hbm, idx_all, *scr):
    rows_v = scr[:NBUF]
    gsem = scr[NBUF:2 * NBUF]
    wsem = scr[2 * NBUF:3 * NBUF]
    isem = scr[3 * NBUF]
    wid = lax.axis_index("s") * NC + lax.axis_index("c")
    base = wid * b_per_w

    pltpu.async_copy(idx_hbm.at[pl.ds(base, b_per_w)], idx_all, isem).wait()

    def idx_ref(j):
      return idx_all.at[pl.ds(j * C, C)]

    def fire_gather(b, j):
      pltpu.async_copy(table_hbm.at[idx_ref(j)], rows_v[b], gsem[b])

    def wait_gather(b, j):
      pltpu.make_async_copy(table_hbm.at[idx_ref(j)], rows_v[b],
                            gsem[b]).wait()

    def fire_write(b, j):
      pltpu.async_copy(rows_v[b], out_hbm.at[pl.ds(base + j * C, C)], wsem[b])

    def wait_write(b):
      pltpu.make_async_copy(rows_v[b], out_hbm.at[pl.ds(0, C)], wsem[b]).wait()

    for j in range(K):
      fire_gather(j, j)

    def round_body(i, carry):
      for b in range(NBUF):
        j = i * NBUF + b
        bp = (b + K) % NBUF
        @pl.when(jnp.logical_and(j >= NBUF - K, j + K < n_chunks))
        def _():
          wait_write(bp)

        @pl.when(j + K < n_chunks)
        def _():
          fire_gather(bp, j + K)

        wait_gather(b, j)
        fire_write(b, j)
      return carry

    lax.fori_loop(0, n_rounds, round_body, 0)

    for t in range(n_tail):
      j = n_rounds * NBUF + t
      b = j % NBUF
      wait_gather(b, j)
      fire_write(b, j)

    for b in range(NBUF):
      wait_write(b)

  return k


def kernel(x, index):
  V, D = x.shape
  B = index.shape[0]
  return _make_gather(V, D, B)(x, index.astype(jnp.int32))


# speedup vs baseline: 1.0207x; 1.0207x over previous
import functools

import jax
import jax.numpy as jnp
from jax import lax
from jax.experimental import pallas as pl
from jax.experimental.pallas import tpu as pltpu, tpu_sc as plsc


def _make_gather(V, D, B):
  info = plsc.get_sparse_core_info()
  NC, NS = info.num_cores, info.num_subcores
  NW = NC * NS
  assert B % NW == 0
  b_per_w = B // NW
  C = 200
  NBUF = 3
  NBUF_S = 2
  K = 2
  assert b_per_w % C == 0
  n_chunks = b_per_w // C
  UNROLL = 6
  n_rounds = n_chunks // UNROLL
  n_tail = n_chunks % UNROLL

  mesh = plsc.VectorSubcoreMesh(core_axis_name="c", subcore_axis_name="s")

  scratch = ([pltpu.VMEM((C,), jnp.int32)] * NBUF
             + [pltpu.VMEM((C, D), jnp.float32)] * NBUF
             + [pltpu.VMEM_SHARED((NS, NBUF_S, C, D), jnp.float32)]
             + [pltpu.SemaphoreType.DMA] * (NBUF + 2 * NBUF_S))

  @functools.partial(
      pl.kernel,
      mesh=mesh,
      out_type=jax.ShapeDtypeStruct((B, D), jnp.float32),
      scratch_types=scratch,
  )
  def k(table_hbm, idx_hbm, out_hbm, *scr):
    idx_v = scr[:NBUF]
    rows_v = scr[NBUF:2 * NBUF]
    rows_s = scr[2 * NBUF]
    gsem = scr[2 * NBUF + 1:3 * NBUF + 1]
    c1sem = scr[3 * NBUF + 1:3 * NBUF + 1 + NBUF_S]
    c2sem = scr[3 * NBUF + 1 + NBUF_S:3 * NBUF + 1 + 2 * NBUF_S]
    cid = lax.axis_index("c")
    sid = lax.axis_index("s")
    wid = sid * NC + cid
    base = wid * b_per_w

    def sbuf(bs):
      return rows_s.at[sid, bs]

    def fire_gather(b, j):
      off = base + j * C
      pltpu.sync_copy(idx_hbm.at[pl.ds(off, C)], idx_v[b])
      pltpu.async_copy(table_hbm.at[idx_v[b]], rows_v[b], gsem[b])

    def wait_gather(b):
      pltpu.make_async_copy(table_hbm.at[idx_v[b]], rows_v[b],
                            gsem[b]).wait()

    def fire_copy1(b, bs):
      pltpu.async_copy(rows_v[b], sbuf(bs), c1sem[bs])

    def wait_copy1(b, bs):
      pltpu.make_async_copy(rows_v[b], sbuf(bs), c1sem[bs]).wait()

    def fire_copy2(bs, j):
      pltpu.async_copy(sbuf(bs), out_hbm.at[pl.ds(base + j * C, C)],
                       c2sem[bs])

    def wait_copy2(bs):
      pltpu.make_async_copy(sbuf(bs), out_hbm.at[pl.ds(0, C)],
                            c2sem[bs]).wait()

    for j in range(K):
      fire_gather(j, j)

    def step(j, b, bs):
      bm = (b - 1) % NBUF
      bsm = (bs - 1) % NBUF_S

      @pl.when(j >= 1)
      def _():
        wait_copy1(bm, bsm)

      @pl.when(j + K < n_chunks)
      def _():
        fire_gather((b + K) % NBUF, j + K)

      @pl.when(j >= NBUF_S)
      def _():
        wait_copy2(bs)

      wait_gather(b)
      fire_copy1(b, bs)

      @pl.when(j >= 1)
      def _():
        fire_copy2(bsm, j - 1)

    def round_body(i, carry):
      for u in range(UNROLL):
        step(i * UNROLL + u, u % NBUF, u % NBUF_S)
      return carry

    lax.fori_loop(0, n_rounds, round_body, 0)

    for t in range(n_tail):
      j = n_rounds * UNROLL + t
      step(j, j % NBUF, j % NBUF_S)

    last = n_chunks - 1
    lb = last % NBUF
    lbs = last % NBUF_S
    wait_copy1(lb, lbs)
    fire_copy2(lbs, last)
    for m in range(NBUF_S):
      wait_copy2((lbs - m) % NBUF_S)

  return k


def kernel(x, index):
  V, D = x.shape
  B = index.shape[0]
  return _make_gather(V, D, B)(x, index.astype(jnp.int32))


# speedup vs baseline: 1.0215x; 1.0007x over previous
import functools

import jax
import jax.numpy as jnp
from jax import lax
from jax.experimental import pallas as pl
from jax.experimental.pallas import tpu as pltpu, tpu_sc as plsc


def _make_gather(V, D, B):
  info = plsc.get_sparse_core_info()
  NC, NS = info.num_cores, info.num_subcores
  NW = NC * NS
  assert B % NW == 0
  b_per_w = B // NW
  C = 200
  NBUF = 3
  NBUF_S = 2
  K = 2
  assert b_per_w % C == 0
  n_chunks = b_per_w // C
  UNROLL = 6
  n_rounds = n_chunks // UNROLL
  n_tail = n_chunks % UNROLL

  mesh = plsc.VectorSubcoreMesh(core_axis_name="c", subcore_axis_name="s")

  scratch = ([pltpu.VMEM((C,), jnp.int32)] * NBUF
             + [pltpu.VMEM((C, D), jnp.float32)] * NBUF
             + [pltpu.VMEM_SHARED((NS, NBUF_S, C, D), jnp.float32)]
             + [pltpu.SemaphoreType.DMA] * (NBUF + 2 * NBUF_S))

  @functools.partial(
      pl.kernel,
      mesh=mesh,
      out_type=jax.ShapeDtypeStruct((B, D), jnp.float32),
      scratch_types=scratch,
  )
  def k(table_hbm, idx_hbm, out_hbm, *scr):
    idx_v = scr[:NBUF]
    rows_v = scr[NBUF:2 * NBUF]
    rows_s = scr[2 * NBUF]
    gsem = scr[2 * NBUF + 1:3 * NBUF + 1]
    c1sem = scr[3 * NBUF + 1:3 * NBUF + 1 + NBUF_S]
    c2sem = scr[3 * NBUF + 1 + NBUF_S:3 * NBUF + 1 + 2 * NBUF_S]
    cid = lax.axis_index("c")
    sid = lax.axis_index("s")
    wid = sid * NC + cid
    base = wid * b_per_w

    def sbuf(bs):
      return rows_s.at[sid, bs]

    def fire_gather(b, j):
      off = base + j * C
      pltpu.sync_copy(idx_hbm.at[pl.ds(off, C)], idx_v[b])
      pltpu.async_copy(table_hbm.at[idx_v[b]], rows_v[b], gsem[b])

    def wait_gather(b):
      pltpu.make_async_copy(table_hbm.at[idx_v[b]], rows_v[b],
                            gsem[b]).wait()

    def fire_copy1(b, bs):
      pltpu.async_copy(rows_v[b], sbuf(bs), c1sem[bs])

    def wait_copy1(b, bs):
      pltpu.make_async_copy(rows_v[b], sbuf(bs), c1sem[bs]).wait()

    def fire_copy2(bs, j):
      pltpu.async_copy(sbuf(bs), out_hbm.at[pl.ds(base + j * C, C)],
                       c2sem[bs])

    def wait_copy2(bs):
      pltpu.make_async_copy(sbuf(bs), out_hbm.at[pl.ds(0, C)],
                            c2sem[bs]).wait()

    for j in range(K):
      fire_gather(j, j)

    def step(j, b, bs):
      bm = (b - 1) % NBUF
      bsm = (bs - 1) % NBUF_S

      @pl.when(j >= 1)
      def _():
        wait_copy1(bm, bsm)

      @pl.when(j + K < n_chunks)
      def _():
        fire_gather((b + K) % NBUF, j + K)

      @pl.when(j >= 1)
      def _():
        fire_copy2(bsm, j - 1)

      @pl.when(j >= NBUF_S)
      def _():
        wait_copy2(bs)

      wait_gather(b)
      fire_copy1(b, bs)

    def round_body(i, carry):
      for u in range(UNROLL):
        step(i * UNROLL + u, u % NBUF, u % NBUF_S)
      return carry

    lax.fori_loop(0, n_rounds, round_body, 0)

    for t in range(n_tail):
      j = n_rounds * UNROLL + t
      step(j, j % NBUF, j % NBUF_S)

    last = n_chunks - 1
    lb = last % NBUF
    lbs = last % NBUF_S
    wait_copy1(lb, lbs)
    fire_copy2(lbs, last)
    for m in range(NBUF_S):
      wait_copy2((lbs - m) % NBUF_S)

  return k


def kernel(x, index):
  V, D = x.shape
  B = index.shape[0]
  return _make_gather(V, D, B)(x, index.astype(jnp.int32))
